# baseline (speedup 1.0000x reference)
import functools
import math

import jax
import jax.numpy as jnp
from jax.experimental import pallas as pl
from jax.experimental.pallas import tpu as pltpu

_HEADS = 16
_HEADS_K = 4
_GROUP = _HEADS // _HEADS_K


def _fused_layer_kernel(h_ref, wq_ref, wk_ref, wv_ref, wo_ref,
                        bq_ref, bk_ref, bv_ref, bo_ref, o_ref, *, scale):
    S = h_ref.shape[0]
    D = wk_ref.shape[1] // _HEADS_K
    GD = _GROUP * D

    x = h_ref[...]
    k = jnp.dot(x, wk_ref[...],
                preferred_element_type=jnp.float32) + bk_ref[...]
    v = (jnp.dot(x.astype(jnp.bfloat16), wv_ref[...],
                 preferred_element_type=jnp.float32)
         + bv_ref[...]).astype(jnp.bfloat16)

    o_ref[...] = jnp.broadcast_to(bo_ref[...], o_ref.shape)

    for hk in range(_HEADS_K):
        q_g = (jnp.dot(x, wq_ref[:, hk * GD:(hk + 1) * GD],
                       preferred_element_type=jnp.float32) * scale
               + bq_ref[:, hk * GD:(hk + 1) * GD])
        q_blk = jnp.concatenate(
            [q_g[:, g * D:(g + 1) * D] for g in range(_GROUP)], axis=0)
        k_h = k[:, hk * D:(hk + 1) * D]
        v_h = v[:, hk * D:(hk + 1) * D]
        s = jax.lax.dot_general(q_blk, k_h, (((1,), (1,)), ((), ())),
                                preferred_element_type=jnp.float32)
        m = s.max(axis=-1, keepdims=True)
        p = jnp.exp(s - m)
        l = p.sum(axis=-1, keepdims=True)
        pv = jnp.dot(p.astype(jnp.bfloat16), v_h,
                     preferred_element_type=jnp.float32)
        o_blk = (pv / l).astype(jnp.bfloat16)
        ao_hk = jnp.concatenate(
            [o_blk[g * S:(g + 1) * S, :] for g in range(_GROUP)], axis=1)
        o_ref[...] += jnp.dot(ao_hk, wo_ref[hk * GD:(hk + 1) * GD, :],
                              preferred_element_type=jnp.float32)


def kernel(h, wq_t, bq, wk_t, bk, wv_t, bv, wo_t, bo):
    B, S, hidden = h.shape
    head_dim = hidden // _HEADS
    dkv = _HEADS_K * head_dim
    scale = 1.0 / math.sqrt(head_dim)
    M = B * S

    h2 = h.reshape(M, hidden)
    wv = wv_t.astype(jnp.bfloat16)
    wo = wo_t.astype(jnp.bfloat16)
    bq2 = (bq * scale).reshape(1, hidden)
    bk2 = bk.reshape(1, dkv)
    bv2 = bv.reshape(1, dkv)
    bo2 = bo.reshape(1, hidden)

    body = functools.partial(_fused_layer_kernel, scale=scale)
    return pl.pallas_call(
        body,
        out_shape=jax.ShapeDtypeStruct((M, hidden), jnp.float32),
        grid=(B,),
        in_specs=[
            pl.BlockSpec((S, hidden), lambda i: (i, 0)),
            pl.BlockSpec(memory_space=pltpu.VMEM),
            pl.BlockSpec(memory_space=pltpu.VMEM),
            pl.BlockSpec(memory_space=pltpu.VMEM),
            pl.BlockSpec(memory_space=pltpu.VMEM),
            pl.BlockSpec(memory_space=pltpu.VMEM),
            pl.BlockSpec(memory_space=pltpu.VMEM),
            pl.BlockSpec(memory_space=pltpu.VMEM),
            pl.BlockSpec(memory_space=pltpu.VMEM),
        ],
        out_specs=pl.BlockSpec((S, hidden), lambda i: (i, 0)),
        compiler_params=pltpu.CompilerParams(
            dimension_semantics=("parallel",),
            vmem_limit_bytes=60 * 1024 * 1024,
        ),
    )(h2, wq_t, wk_t, wv, wo, bq2, bk2, bv2, bo2)


# speedup vs baseline: 1.0265x; 1.0265x over previous
import math

import jax
import jax.numpy as jnp
from jax.experimental import pallas as pl
from jax.experimental.pallas import tpu as pltpu

_HEADS = 16
_HEADS_K = 4
_GROUP = _HEADS // _HEADS_K


def _qkv_attn_kernel(h_ref, wq_ref, wk_ref, wv_ref,
                     bq_ref, bk_ref, bv_ref, ao_ref):
    S = h_ref.shape[0]
    D = wk_ref.shape[1] // _HEADS_K

    x = h_ref[...]
    q = jnp.dot(x, wq_ref[...],
                preferred_element_type=jnp.float32) + bq_ref[...]
    k = jnp.dot(x, wk_ref[...],
                preferred_element_type=jnp.float32) + bk_ref[...]
    v = (jnp.dot(x.astype(jnp.bfloat16), wv_ref[...],
                 preferred_element_type=jnp.float32)
         + bv_ref[...]).astype(jnp.bfloat16)

    for hk in range(_HEADS_K):
        k_h = k[:, hk * D:(hk + 1) * D]
        v_h = v[:, hk * D:(hk + 1) * D]
        q_blk = jnp.concatenate(
            [q[:, (hk * _GROUP + g) * D:(hk * _GROUP + g + 1) * D]
             for g in range(_GROUP)], axis=0)
        s = jax.lax.dot_general(q_blk, k_h, (((1,), (1,)), ((), ())),
                                preferred_element_type=jnp.float32)
        m = s.max(axis=-1, keepdims=True)
        p = jnp.exp(s - m)
        l = p.sum(axis=-1, keepdims=True)
        pv = jnp.dot(p.astype(jnp.bfloat16), v_h,
                     preferred_element_type=jnp.float32)
        o_blk = (pv / l).astype(jnp.bfloat16)
        for g in range(_GROUP):
            h = hk * _GROUP + g
            ao_ref[:, h * D:(h + 1) * D] = o_blk[g * S:(g + 1) * S, :]


def _out_proj_kernel(x_ref, w_ref, b_ref, o_ref):
    o_ref[...] = (jnp.dot(x_ref[...], w_ref[...],
                          preferred_element_type=jnp.float32) + b_ref[...])


def kernel(h, wq_t, bq, wk_t, bk, wv_t, bv, wo_t, bo):
    B, S, hidden = h.shape
    head_dim = hidden // _HEADS
    dkv = _HEADS_K * head_dim
    scale = 1.0 / math.sqrt(head_dim)
    M = B * S

    h2 = h.reshape(M, hidden)
    wq = wq_t * scale
    bq2 = (bq * scale).reshape(1, hidden)
    wv = wv_t.astype(jnp.bfloat16)
    wo = wo_t.astype(jnp.bfloat16)
    bk2 = bk.reshape(1, dkv)
    bv2 = bv.reshape(1, dkv)
    bo2 = bo.reshape(1, hidden)

    ao = pl.pallas_call(
        _qkv_attn_kernel,
        out_shape=jax.ShapeDtypeStruct((M, hidden), jnp.bfloat16),
        grid=(B,),
        in_specs=[
            pl.BlockSpec((S, hidden), lambda i: (i, 0)),
            pl.BlockSpec(memory_space=pltpu.VMEM),
            pl.BlockSpec(memory_space=pltpu.VMEM),
            pl.BlockSpec(memory_space=pltpu.VMEM),
            pl.BlockSpec(memory_space=pltpu.VMEM),
            pl.BlockSpec(memory_space=pltpu.VMEM),
            pl.BlockSpec(memory_space=pltpu.VMEM),
        ],
        out_specs=pl.BlockSpec((S, hidden), lambda i: (i, 0)),
        compiler_params=pltpu.CompilerParams(
            dimension_semantics=("parallel",),
            vmem_limit_bytes=60 * 1024 * 1024,
        ),
    )(h2, wq, wk_t, wv, bq2, bk2, bv2)

    tm = 512
    return pl.pallas_call(
        _out_proj_kernel,
        out_shape=jax.ShapeDtypeStruct((M, hidden), jnp.float32),
        grid=(M // tm,),
        in_specs=[
            pl.BlockSpec((tm, hidden), lambda i: (i, 0)),
            pl.BlockSpec(memory_space=pltpu.VMEM),
            pl.BlockSpec(memory_space=pltpu.VMEM),
        ],
        out_specs=pl.BlockSpec((tm, hidden), lambda i: (i, 0)),
        compiler_params=pltpu.CompilerParams(
            dimension_semantics=("parallel",),
            vmem_limit_bytes=60 * 1024 * 1024,
        ),
    )(ao, wo, bo2)
